# baseline (speedup 1.0000x reference)
import functools

import jax
import jax.numpy as jnp
from jax import lax
from jax.experimental import pallas as pl
from jax.experimental.pallas import tpu as pltpu
from jax.experimental.pallas import tpu_sc as plsc

_BETA = 0.9
_LAMBDA = 7.0
_B = 4096
_C = 128
_BLK = 2048
_STEPS = _B // _BLK

_NC = 2
_NS = 16
_NW = _NC * _NS
_BPW = _B // _NW


_HALF = _BPW // 2


def _sc_gather_body(table_hbm, idx_hbm, out_hbm, idx_v, rows_v,
                    gsem0, gsem1, wsem0, wsem1):
    wid = lax.axis_index("s") * _NC + lax.axis_index("c")
    base = wid * _BPW
    pltpu.sync_copy(idx_hbm.at[pl.ds(base, _HALF)], idx_v.at[0])
    pltpu.sync_copy(idx_hbm.at[pl.ds(base + _HALF, _HALF)], idx_v.at[1])
    cp0 = pltpu.async_copy(table_hbm.at[idx_v.at[0]], rows_v.at[0], gsem0)
    cp1 = pltpu.async_copy(table_hbm.at[idx_v.at[1]], rows_v.at[1], gsem1)
    cp0.wait()
    w0 = pltpu.async_copy(rows_v.at[0], out_hbm.at[pl.ds(base, _HALF)], wsem0)
    cp1.wait()
    w1 = pltpu.async_copy(rows_v.at[1], out_hbm.at[pl.ds(base + _HALF, _HALF)],
                          wsem1)
    w0.wait()
    w1.wait()


def _tc_stage1_body(out_ref, p_ref, t2_ref):
    x = out_ref[...]
    m = jnp.max(x, axis=1, keepdims=True)
    e = jnp.exp(x - m)
    s = jnp.sum(e, axis=1, keepdims=True)
    p = e / s
    p = jnp.clip(p, 0.0001, 1.0 - 0.0001)
    pn = p / jnp.sum(p, axis=1, keepdims=True)
    p_ref[...] = p.astype(jnp.bfloat16)
    t2_ref[...] = jnp.sum(pn * p, axis=1)[None, None, :]


def _tc_stage2_body(closs_ref, old_ref, p_ref, t2_ref, loss_ref, acc_ref):
    i = pl.program_id(0)

    @pl.when(i == 0)
    def _():
        acc_ref[0, 0] = 0.0

    d = _BETA * jnp.sum(old_ref[...] * p_ref[...].astype(jnp.float32), axis=1) \
        + (1.0 - _BETA) * t2_ref[0, 0, :]
    acc_ref[0, 0] += jnp.sum(jnp.log(1.0 - d))

    @pl.when(i == _STEPS - 1)
    def _():
        loss_ref[0, 0] = closs_ref[0] + _LAMBDA * (acc_ref[0, 0] / _B)


def kernel(index, output, label, contrastive_loss, confi_weight, target):
    del label, confi_weight

    mesh = plsc.VectorSubcoreMesh(
        core_axis_name="c", subcore_axis_name="s",
        num_cores=_NC, num_subcores=_NS,
    )
    sc_gather = functools.partial(
        pl.kernel,
        mesh=mesh,
        out_type=jax.ShapeDtypeStruct((_B, _C), jnp.float32),
        scratch_types=[
            pltpu.VMEM((2, _HALF), jnp.int32),
            pltpu.VMEM((2, _HALF, _C), jnp.float32),
            pltpu.SemaphoreType.DMA,
            pltpu.SemaphoreType.DMA,
            pltpu.SemaphoreType.DMA,
            pltpu.SemaphoreType.DMA,
        ],
    )(_sc_gather_body)
    gathered = sc_gather(target, index)

    p, t2 = pl.pallas_call(
        _tc_stage1_body,
        grid=(_STEPS,),
        in_specs=[pl.BlockSpec((_BLK, _C), lambda i: (i, 0))],
        out_specs=[
            pl.BlockSpec((_BLK, _C), lambda i: (i, 0)),
            pl.BlockSpec((1, 1, _BLK), lambda i: (i, 0, 0)),
        ],
        out_shape=[
            jax.ShapeDtypeStruct((_B, _C), jnp.bfloat16),
            jax.ShapeDtypeStruct((_STEPS, 1, _BLK), jnp.float32),
        ],
    )(output)

    closs = jnp.reshape(contrastive_loss, (1,))
    loss = pl.pallas_call(
        _tc_stage2_body,
        grid=(_STEPS,),
        in_specs=[
            pl.BlockSpec(memory_space=pltpu.SMEM),
            pl.BlockSpec((_BLK, _C), lambda i: (i, 0)),
            pl.BlockSpec((_BLK, _C), lambda i: (i, 0)),
            pl.BlockSpec((1, 1, _BLK), lambda i: (i, 0, 0)),
        ],
        out_specs=pl.BlockSpec(memory_space=pltpu.SMEM),
        out_shape=jax.ShapeDtypeStruct((1, 1), jnp.float32),
        scratch_shapes=[pltpu.SMEM((1, 1), jnp.float32)],
    )(closs, gathered, p, t2)
    return jnp.reshape(loss, ())


# speedup vs baseline: 1.0352x; 1.0352x over previous
import functools

import jax
import jax.numpy as jnp
from jax import lax
from jax.experimental import pallas as pl
from jax.experimental.pallas import tpu as pltpu
from jax.experimental.pallas import tpu_sc as plsc

_BETA = 0.9
_LAMBDA = 7.0
_B = 4096
_C = 128
_BLK = 2048
_STEPS = _B // _BLK

_NC = 1
_NS = 16
_NW = _NC * _NS
_BPW = _B // _NW


_CHUNKS = 4
_CROWS = _BPW // _CHUNKS


def _sc_gather_body(table_hbm, idx_hbm, out_hbm, idx_v, rows_v, gsems, wsems):
    wid = lax.axis_index("s") * _NC + lax.axis_index("c")
    base = wid * _BPW
    pltpu.sync_copy(idx_hbm.at[pl.ds(base, _BPW)], idx_v)
    gathers = [
        pltpu.async_copy(
            table_hbm.at[idx_v.at[pl.ds(c * _CROWS, _CROWS)]],
            rows_v.at[c], gsems[c])
        for c in range(_CHUNKS)
    ]
    writes = []
    for c in range(_CHUNKS):
        gathers[c].wait()
        writes.append(pltpu.async_copy(
            rows_v.at[c], out_hbm.at[pl.ds(base + c * _CROWS, _CROWS)],
            wsems[c]))
    for w in writes:
        w.wait()


def _tc_stage1_body(out_ref, p_ref, t2_ref):
    x = out_ref[...]
    m = jnp.max(x, axis=1, keepdims=True)
    e = jnp.exp(x - m)
    s = jnp.sum(e, axis=1, keepdims=True)
    p = e / s
    p = jnp.clip(p, 0.0001, 1.0 - 0.0001)
    pn = p / jnp.sum(p, axis=1, keepdims=True)
    p_ref[...] = p.astype(jnp.bfloat16)
    t2_ref[...] = jnp.sum(pn * p, axis=1)[None, None, :]


def _tc_stage2_body(closs_ref, old_ref, p_ref, t2_ref, loss_ref, acc_ref):
    i = pl.program_id(0)

    @pl.when(i == 0)
    def _():
        acc_ref[0, 0] = 0.0

    d = _BETA * jnp.sum(old_ref[...] * p_ref[...].astype(jnp.float32), axis=1) \
        + (1.0 - _BETA) * t2_ref[0, 0, :]
    acc_ref[0, 0] += jnp.sum(jnp.log(1.0 - d))

    @pl.when(i == _STEPS - 1)
    def _():
        loss_ref[0, 0] = closs_ref[0] + _LAMBDA * (acc_ref[0, 0] / _B)


def kernel(index, output, label, contrastive_loss, confi_weight, target):
    del label, confi_weight

    mesh = plsc.VectorSubcoreMesh(
        core_axis_name="c", subcore_axis_name="s",
        num_cores=_NC, num_subcores=_NS,
    )
    sc_gather = functools.partial(
        pl.kernel,
        mesh=mesh,
        out_type=jax.ShapeDtypeStruct((_B, _C), jnp.float32),
        scratch_types=[
            pltpu.VMEM((_BPW,), jnp.int32),
            pltpu.VMEM((_CHUNKS, _CROWS, _C), jnp.float32),
            [pltpu.SemaphoreType.DMA] * _CHUNKS,
            [pltpu.SemaphoreType.DMA] * _CHUNKS,
        ],
    )(_sc_gather_body)
    gathered = sc_gather(target, index)

    p, t2 = pl.pallas_call(
        _tc_stage1_body,
        grid=(_STEPS,),
        in_specs=[pl.BlockSpec((_BLK, _C), lambda i: (i, 0))],
        out_specs=[
            pl.BlockSpec((_BLK, _C), lambda i: (i, 0)),
            pl.BlockSpec((1, 1, _BLK), lambda i: (i, 0, 0)),
        ],
        out_shape=[
            jax.ShapeDtypeStruct((_B, _C), jnp.bfloat16),
            jax.ShapeDtypeStruct((_STEPS, 1, _BLK), jnp.float32),
        ],
    )(output)

    closs = jnp.reshape(contrastive_loss, (1,))
    loss = pl.pallas_call(
        _tc_stage2_body,
        grid=(_STEPS,),
        in_specs=[
            pl.BlockSpec(memory_space=pltpu.SMEM),
            pl.BlockSpec((_BLK, _C), lambda i: (i, 0)),
            pl.BlockSpec((_BLK, _C), lambda i: (i, 0)),
            pl.BlockSpec((1, 1, _BLK), lambda i: (i, 0, 0)),
        ],
        out_specs=pl.BlockSpec(memory_space=pltpu.SMEM),
        out_shape=jax.ShapeDtypeStruct((1, 1), jnp.float32),
        scratch_shapes=[pltpu.SMEM((1, 1), jnp.float32)],
    )(closs, gathered, p, t2)
    return jnp.reshape(loss, ())
